# baseline (speedup 1.0000x reference)
import functools

import numpy as np

import jax
import jax.numpy as jnp
from jax.experimental import pallas as pl
from jax.experimental.pallas import tpu as pltpu
from jax.sharding import Mesh, PartitionSpec as P

HIDDEN = 4096
NUM_EXPERTS = 64
TOP_K = 8
TOKEN_BLOCK = 512
CHUNK = 256
NBUF = 4


def _start_copy(x_hbm, xbuf, sems, step):
    slot = jax.lax.rem(step, NBUF)
    pltpu.make_async_copy(
        x_hbm.at[pl.ds(step * TOKEN_BLOCK, TOKEN_BLOCK), :],
        xbuf.at[slot],
        sems.at[slot],
    ).start()


def _router_kernel(x_hbm, w_ref, topw_ref, topi_ref, zsum_ref, counts_ref,
                   probsum_ref, xbuf, sems, *, num_steps):
    i = pl.program_id(0)

    @pl.when(i == 0)
    def _init():
        zsum_ref[...] = jnp.zeros_like(zsum_ref)
        counts_ref[...] = jnp.zeros_like(counts_ref)
        probsum_ref[...] = jnp.zeros_like(probsum_ref)
        for k in range(NBUF - 1):
            _start_copy(x_hbm, xbuf, sems, jnp.int32(k))

    @pl.when(i + NBUF - 1 < num_steps)
    def _prefetch():
        _start_copy(x_hbm, xbuf, sems, i + NBUF - 1)

    slot = jax.lax.rem(i, NBUF)
    pltpu.make_async_copy(
        x_hbm.at[pl.ds(i * TOKEN_BLOCK, TOKEN_BLOCK), :],
        xbuf.at[slot],
        sems.at[slot],
    ).wait()

    w = w_ref[...]
    acc_z = jnp.zeros((1, 1), jnp.float32)
    acc_probsum = jnp.zeros((NUM_EXPERTS, 1), jnp.float32)
    acc_counts = jnp.zeros((NUM_EXPERTS, 1), jnp.float32)

    for c in range(TOKEN_BLOCK // CHUNK):
        sl = pl.ds(c * CHUNK, CHUNK)
        x = xbuf[slot, sl, :]
        lt = jax.lax.dot_general(
            w, x, (((1,), (1,)), ((), ())), preferred_element_type=jnp.float32)

        m = jnp.max(lt, axis=0, keepdims=True)
        e = jnp.exp(lt - m)
        s = jnp.sum(e, axis=0, keepdims=True)
        probs = e / s

        lse = m + jnp.log(s)
        acc_z += jnp.sum(lse * lse).reshape(1, 1)

        acc_probsum += jnp.sum(probs, axis=1, keepdims=True)

        iota = jax.lax.broadcasted_iota(jnp.int32, probs.shape, 0)
        bits = jax.lax.bitcast_convert_type(probs, jnp.int32)
        key = (bits & ~63) | (NUM_EXPERTS - 1 - iota)
        top_keys = []
        for _ in range(TOP_K):
            cur = jnp.max(key, axis=0, keepdims=True)
            top_keys.append(cur)
            key = jnp.where(key == cur, jnp.int32(-2**31), key)

        topk = jnp.concatenate(top_keys, axis=0)
        topi = (NUM_EXPERTS - 1) - (topk & 63)
        topw = jax.lax.bitcast_convert_type(topk & ~63, jnp.float32)
        topw = topw / jnp.sum(topw, axis=0, keepdims=True)
        topw_ref[sl, :] = topw.T
        topi_ref[sl, :] = topi.T

        top1_idx = topi[0:1, :]
        acc_counts += jnp.sum((iota == top1_idx).astype(jnp.float32), axis=1,
                              keepdims=True)

    zsum_ref[...] += acc_z
    counts_ref[...] += acc_counts
    probsum_ref[...] += acc_probsum


def _router_shard(x, W):
    n = x.shape[0]
    num_steps = n // TOKEN_BLOCK
    kern = functools.partial(_router_kernel, num_steps=num_steps)
    return pl.pallas_call(
        kern,
        grid=(num_steps,),
        in_specs=[
            pl.BlockSpec(memory_space=pl.ANY),
            pl.BlockSpec((NUM_EXPERTS, HIDDEN), lambda i: (0, 0)),
        ],
        out_specs=[
            pl.BlockSpec((TOKEN_BLOCK, TOP_K), lambda i: (i, 0)),
            pl.BlockSpec((TOKEN_BLOCK, TOP_K), lambda i: (i, 0)),
            pl.BlockSpec((1, 1), lambda i: (0, 0)),
            pl.BlockSpec((NUM_EXPERTS, 1), lambda i: (0, 0)),
            pl.BlockSpec((NUM_EXPERTS, 1), lambda i: (0, 0)),
        ],
        out_shape=[
            jax.ShapeDtypeStruct((n, TOP_K), jnp.float32),
            jax.ShapeDtypeStruct((n, TOP_K), jnp.int32),
            jax.ShapeDtypeStruct((1, 1), jnp.float32),
            jax.ShapeDtypeStruct((NUM_EXPERTS, 1), jnp.float32),
            jax.ShapeDtypeStruct((NUM_EXPERTS, 1), jnp.float32),
        ],
        scratch_shapes=[
            pltpu.VMEM((NBUF, TOKEN_BLOCK, HIDDEN), jnp.float32),
            pltpu.SemaphoreType.DMA((NBUF,)),
        ],
        compiler_params=pltpu.CompilerParams(
            dimension_semantics=("arbitrary",)),
    )(x, W)


def kernel(hidden_states, W):
    B, S, H = hidden_states.shape
    x = hidden_states.reshape(-1, H)
    num_tokens = x.shape[0]

    devs = jax.devices()
    n_shards = 2 if (len(devs) >= 2 and num_tokens % (2 * TOKEN_BLOCK) == 0) \
        else 1
    if n_shards > 1:
        mesh = Mesh(np.array(devs[:n_shards]), ("t",))
        sharded = jax.shard_map(
            _router_shard, mesh=mesh,
            in_specs=(P("t", None), P(None, None)),
            out_specs=(P("t", None), P("t", None), P("t", None),
                       P("t", None), P("t", None)),
            check_vma=False,
        )
        topw, topi, zsum, counts, probsum = sharded(x, W)
        zsum = jnp.sum(zsum.reshape(n_shards, 1), axis=0)
        counts = jnp.sum(counts.reshape(n_shards, NUM_EXPERTS), axis=0)
        probsum = jnp.sum(probsum.reshape(n_shards, NUM_EXPERTS), axis=0)
    else:
        topw, topi, zsum, counts, probsum = _router_shard(x, W)
        counts = counts.reshape(NUM_EXPERTS)
        probsum = probsum.reshape(NUM_EXPERTS)

    inv_n = 1.0 / num_tokens
    lbl = (NUM_EXPERTS * inv_n * inv_n) * jnp.sum(counts * probsum)
    zl = jnp.sum(zsum) * inv_n
    util = counts * inv_n
    return (topw, topi, lbl.reshape(()), zl.reshape(()), util)


# speedup vs baseline: 6.1584x; 6.1584x over previous
import functools

import jax
import jax.numpy as jnp
from jax.experimental import pallas as pl
from jax.experimental.pallas import tpu as pltpu

HIDDEN = 4096
NUM_EXPERTS = 64
TOP_K = 8
TOKEN_BLOCK = 1024
CHUNK = 256


def _router_kernel(x_ref, w_ref, topw_ref, topi_ref, lbl_ref, zl_ref,
                   util_ref, probsum_ref, *, num_tokens, num_steps):
    i = pl.program_id(0)

    @pl.when(i == 0)
    def _init():
        zl_ref[...] = jnp.zeros_like(zl_ref)
        util_ref[...] = jnp.zeros_like(util_ref)
        probsum_ref[...] = jnp.zeros_like(probsum_ref)
        lbl_ref[...] = jnp.zeros_like(lbl_ref)

    w = w_ref[...]
    acc_z = jnp.zeros((1, 1), jnp.float32)
    acc_probsum = jnp.zeros((NUM_EXPERTS, 1), jnp.float32)
    acc_counts = jnp.zeros((NUM_EXPERTS, 1), jnp.float32)

    for c in range(TOKEN_BLOCK // CHUNK):
        sl = pl.ds(c * CHUNK, CHUNK)
        x = x_ref[sl, :]
        lt = jax.lax.dot_general(
            w, x, (((1,), (1,)), ((), ())), preferred_element_type=jnp.float32)

        m = jnp.max(lt, axis=0, keepdims=True)
        e = jnp.exp(lt - m)
        s = jnp.sum(e, axis=0, keepdims=True)
        probs = e / s

        lse = m + jnp.log(s)
        acc_z += jnp.sum(lse * lse).reshape(1, 1)

        acc_probsum += jnp.sum(probs, axis=1, keepdims=True)

        iota = jax.lax.broadcasted_iota(jnp.int32, probs.shape, 0)
        bits = jax.lax.bitcast_convert_type(probs, jnp.int32)
        key = (bits & ~63) | (NUM_EXPERTS - 1 - iota)
        top_keys = []
        for _ in range(TOP_K):
            cur = jnp.max(key, axis=0, keepdims=True)
            top_keys.append(cur)
            key = jnp.where(key == cur, jnp.int32(-2**31), key)

        topk = jnp.concatenate(top_keys, axis=0)
        topi = (NUM_EXPERTS - 1) - (topk & 63)
        topw = jax.lax.bitcast_convert_type(topk & ~63, jnp.float32)
        topw = topw / jnp.sum(topw, axis=0, keepdims=True)
        topw_ref[sl, :] = topw.T
        topi_ref[sl, :] = topi.T

        top1_idx = topi[0:1, :]
        acc_counts += jnp.sum((iota == top1_idx).astype(jnp.float32), axis=1,
                              keepdims=True)

    zl_ref[...] += acc_z
    probsum_ref[...] += acc_probsum
    util_ref[...] += acc_counts

    @pl.when(i == num_steps - 1)
    def _finalize():
        counts = util_ref[...]
        probsum = probsum_ref[...]
        inv_n = 1.0 / num_tokens
        lbl_ref[...] = ((NUM_EXPERTS * inv_n * inv_n)
                        * jnp.sum(counts * probsum)).reshape(1, 1)
        zl_ref[...] = zl_ref[...] * inv_n
        util_ref[...] = counts * inv_n


def kernel(hidden_states, W):
    B, S, H = hidden_states.shape
    x = hidden_states.reshape(-1, H)
    num_tokens = x.shape[0]
    num_steps = num_tokens // TOKEN_BLOCK

    grid = (num_steps,)
    kern = functools.partial(_router_kernel, num_tokens=num_tokens,
                             num_steps=num_steps)
    topw, topi, lbl, zl, util = pl.pallas_call(
        kern,
        grid=grid,
        in_specs=[
            pl.BlockSpec((TOKEN_BLOCK, H), lambda i: (i, 0)),
            pl.BlockSpec((NUM_EXPERTS, H), lambda i: (0, 0)),
        ],
        out_specs=[
            pl.BlockSpec((TOKEN_BLOCK, TOP_K), lambda i: (i, 0)),
            pl.BlockSpec((TOKEN_BLOCK, TOP_K), lambda i: (i, 0)),
            pl.BlockSpec((1, 1), lambda i: (0, 0)),
            pl.BlockSpec((1, 1), lambda i: (0, 0)),
            pl.BlockSpec((NUM_EXPERTS, 1), lambda i: (0, 0)),
        ],
        out_shape=[
            jax.ShapeDtypeStruct((num_tokens, TOP_K), jnp.float32),
            jax.ShapeDtypeStruct((num_tokens, TOP_K), jnp.int32),
            jax.ShapeDtypeStruct((1, 1), jnp.float32),
            jax.ShapeDtypeStruct((1, 1), jnp.float32),
            jax.ShapeDtypeStruct((NUM_EXPERTS, 1), jnp.float32),
        ],
        scratch_shapes=[pltpu.VMEM((NUM_EXPERTS, 1), jnp.float32)],
        compiler_params=pltpu.CompilerParams(
            dimension_semantics=("arbitrary",)),
    )(x, W)

    return (topw, topi, lbl.reshape(()), zl.reshape(()), util.reshape(-1))
